# baseline (speedup 1.0000x reference)
import functools

import jax
import jax.numpy as jnp
from jax import lax
from jax.experimental import pallas as pl
from jax.experimental.pallas import tpu as pltpu
from jax.experimental.pallas import tpu_sc as plsc

_P = 0.5
_B, _C, _T = 32, 128, 8192
_L = 16


def _indices(B, T):
    prob = jnp.array([_P / 2.0, 1.0 - _P, _P / 2.0], dtype=jnp.float32)
    key = jax.random.key(42)
    d = jax.random.categorical(key, jnp.log(prob), shape=(B, T)) - 1
    d = d.at[:, 0].set(jnp.clip(d[:, 0], 0, 1))
    d = d.at[:, -1].set(jnp.clip(d[:, -1], -1, 0))
    return (d + jnp.arange(T, dtype=d.dtype)).astype(jnp.int32)


_mesh = plsc.VectorSubcoreMesh(core_axis_name="c", subcore_axis_name="s")


_G = 4
_H = 2
_CSC = 56
_CTC = _C - _CSC
_NG = _CSC // _G


@functools.partial(
    pl.kernel,
    mesh=_mesh,
    out_type=jax.ShapeDtypeStruct((_B, _CSC, _T), jnp.float32),
    scratch_types=[
        pltpu.VMEM((_T,), jnp.int32),
        pltpu.VMEM((_G, _T), jnp.float32),
        pltpu.VMEM((_G, _T), jnp.float32),
        pltpu.VMEM((_H, _T), jnp.float32),
        pltpu.VMEM((_H, _T), jnp.float32),
        pltpu.SemaphoreType.DMA,
        pltpu.SemaphoreType.DMA,
        pltpu.SemaphoreType.DMA,
        pltpu.SemaphoreType.DMA,
    ],
    compiler_params=pltpu.CompilerParams(needs_layout_passes=False),
)
def _jitter_sc(x_hbm, idx_hbm, out_hbm, idx_v, row0, row1, ohalf0, ohalf1,
               in0, in1, out0, out1):
    b = lax.axis_index("s") * 2 + lax.axis_index("c")
    pltpu.sync_copy(idx_hbm.at[b], idx_v)

    rows = (row0, row1)
    ohalves = (ohalf0, ohalf1)
    in_sems = (in0, in1)
    out_sems = (out0, out1)

    def gather_half(rb, ob, r0):
        @plsc.parallel_loop(0, _T // _L, unroll=8)
        def _(o):
            iv = idx_v[pl.ds(o * _L, _L)]
            for r in range(_H):
                rv = jnp.full((_L,), r0 + r, jnp.int32)
                ob[r, pl.ds(o * _L, _L)] = plsc.load_gather(rb, [rv, iv])

    pltpu.async_copy(x_hbm.at[b, pl.ds(0, _G)], row0, in0)

    def pair_body(p, carry):
        g0 = 2 * p
        for k in range(2):
            g = g0 + k
            rb = rows[k]
            c = g * _G
            pltpu.make_async_copy(x_hbm.at[b, pl.ds(c, _G)], rb,
                                  in_sems[k]).wait()
            @pl.when(g + 1 < _NG)
            def _():
                nk = 1 - k
                pltpu.async_copy(x_hbm.at[b, pl.ds(c + _G, _G)],
                                 rows[nk], in_sems[nk])
            for h in range(_G // _H):
                ob = ohalves[h]
                @pl.when(g > 0)
                def _():
                    pltpu.make_async_copy(
                        ob, out_hbm.at[b, pl.ds(c - _G + h * _H, _H)],
                        out_sems[h]).wait()
                gather_half(rb, ob, h * _H)
                pltpu.async_copy(ob, out_hbm.at[b, pl.ds(c + h * _H, _H)],
                                 out_sems[h])
        return carry

    lax.fori_loop(0, _NG // 2, pair_body, 0)
    clast = (_NG - 1) * _G
    pltpu.make_async_copy(ohalf0, out_hbm.at[b, pl.ds(clast, _H)],
                          out0).wait()
    pltpu.make_async_copy(ohalf1, out_hbm.at[b, pl.ds(clast + _H, _H)],
                          out1).wait()


def _jitter_tc_body(d_ref, x_ref, o_ref):
    xb = x_ref[0]
    d = d_ref[0]
    xm = jnp.concatenate([xb[:, :1], xb[:, :-1]], axis=1)
    xp = jnp.concatenate([xb[:, 1:], xb[:, -1:]], axis=1)
    o_ref[0] = jnp.where(d < 0, xm, jnp.where(d > 0, xp, xb))


def kernel(x):
    idx = _indices(_B, _T)
    d = (idx - jnp.arange(_T, dtype=jnp.int32)).reshape(_B, 1, _T)
    out_sc = _jitter_sc(x, idx)
    cb0 = _CSC // 8
    out_tc = pl.pallas_call(
        _jitter_tc_body,
        grid=(_B, _CTC // 8),
        in_specs=[
            pl.BlockSpec((1, 1, _T), lambda b, j: (b, 0, 0)),
            pl.BlockSpec((1, 8, _T), lambda b, j: (b, cb0 + j, 0)),
        ],
        out_specs=pl.BlockSpec((1, 8, _T), lambda b, j: (b, j, 0)),
        out_shape=jax.ShapeDtypeStruct((_B, _CTC, _T), jnp.float32),
    )(d, x)
    return jnp.concatenate([out_sc, out_tc], axis=1)


# speedup vs baseline: 2.5778x; 2.5778x over previous
import functools

import jax
import jax.numpy as jnp
from jax import lax
from jax.experimental import pallas as pl
from jax.experimental.pallas import tpu as pltpu
from jax.experimental.pallas import tpu_sc as plsc

_P = 0.5
_B, _C, _T = 32, 128, 8192
_L = 16


def _indices(B, T):
    prob = jnp.array([_P / 2.0, 1.0 - _P, _P / 2.0], dtype=jnp.float32)
    key = jax.random.key(42)
    d = jax.random.categorical(key, jnp.log(prob), shape=(B, T)) - 1
    d = d.at[:, 0].set(jnp.clip(d[:, 0], 0, 1))
    d = d.at[:, -1].set(jnp.clip(d[:, -1], -1, 0))
    return (d + jnp.arange(T, dtype=d.dtype)).astype(jnp.int32)


_mesh = plsc.VectorSubcoreMesh(core_axis_name="c", subcore_axis_name="s")


_G = 4
_H = 2
_NG = _C // _G


@functools.partial(
    pl.kernel,
    mesh=_mesh,
    out_type=jax.ShapeDtypeStruct((_B, _C, _T), jnp.float32),
    scratch_types=[
        pltpu.VMEM((_T,), jnp.int32),
        pltpu.VMEM((_G, _T), jnp.float32),
        pltpu.VMEM((_G, _T), jnp.float32),
        pltpu.VMEM((_H, _T), jnp.float32),
        pltpu.VMEM((_H, _T), jnp.float32),
        pltpu.SemaphoreType.DMA,
        pltpu.SemaphoreType.DMA,
        pltpu.SemaphoreType.DMA,
        pltpu.SemaphoreType.DMA,
    ],
    compiler_params=pltpu.CompilerParams(needs_layout_passes=False),
)
def _jitter_sc(x_hbm, idx_hbm, out_hbm, idx_v, row0, row1, ohalf0, ohalf1,
               in0, in1, out0, out1):
    b = lax.axis_index("s") * 2 + lax.axis_index("c")
    pltpu.sync_copy(idx_hbm.at[b], idx_v)

    rows = (row0, row1)
    ohalves = (ohalf0, ohalf1)
    in_sems = (in0, in1)
    out_sems = (out0, out1)

    def gather_half(rb, ob, r0):
        @plsc.parallel_loop(0, _T // _L, unroll=8)
        def _(o):
            iv = idx_v[pl.ds(o * _L, _L)]
            for r in range(_H):
                rv = jnp.full((_L,), r0 + r, jnp.int32)
                ob[r, pl.ds(o * _L, _L)] = plsc.load_gather(rb, [rv, iv])

    pltpu.async_copy(x_hbm.at[b, pl.ds(0, _G)], row0, in0)

    def pair_body(p, carry):
        g0 = 2 * p
        for k in range(2):
            g = g0 + k
            rb = rows[k]
            c = g * _G
            pltpu.make_async_copy(x_hbm.at[b, pl.ds(c, _G)], rb,
                                  in_sems[k]).wait()
            @pl.when(g + 1 < _NG)
            def _():
                nk = 1 - k
                pltpu.async_copy(x_hbm.at[b, pl.ds(c + _G, _G)],
                                 rows[nk], in_sems[nk])
            for h in range(_G // _H):
                ob = ohalves[h]
                @pl.when(g > 0)
                def _():
                    pltpu.make_async_copy(
                        ob, out_hbm.at[b, pl.ds(c - _G + h * _H, _H)],
                        out_sems[h]).wait()
                gather_half(rb, ob, h * _H)
                pltpu.async_copy(ob, out_hbm.at[b, pl.ds(c + h * _H, _H)],
                                 out_sems[h])
        return carry

    lax.fori_loop(0, _NG // 2, pair_body, 0)
    clast = (_NG - 1) * _G
    pltpu.make_async_copy(ohalf0, out_hbm.at[b, pl.ds(clast, _H)],
                          out0).wait()
    pltpu.make_async_copy(ohalf1, out_hbm.at[b, pl.ds(clast + _H, _H)],
                          out1).wait()


def kernel(x):
    idx = _indices(_B, _T)
    return _jitter_sc(x, idx)
